# baseline (speedup 1.0000x reference)
import functools

import jax
import jax.numpy as jnp
import numpy as np
from jax import lax
from jax.experimental import pallas as pl
from jax.experimental.pallas import tpu as pltpu
from jax.experimental.pallas import tpu_sc as plsc

_EPS_FLOOR = float(np.finfo(float).eps)

T_TOK = 4096
N_C = 384
N_E = 8
N_HID = 1536
RBLK = 512
N_RBLK = T_TOK // RBLK
MBLK = 512
N_MBLK = 2 * T_TOK // MBLK + N_E
P_ROWS = N_MBLK * MBLK
NW = 32
CHUNK = T_TOK // NW
HALF = CHUNK // 2


def _gelu_exact(v):
    return v * 0.5 * (1.0 + lax.erf(v * np.float32(1.0 / np.sqrt(2.0))))


def _bf16_dot(a, b, dims):
    return lax.dot_general(
        a.astype(jnp.bfloat16), b.astype(jnp.bfloat16), (dims, ((), ())),
        preferred_element_type=jnp.float32)


def _router_kernel(x_ref, pt_ref, w_g_ref,
                   p1_ref, p2_ref, g1_ref, g2_ref, eob_ref, loss_ref,
                   run_scr, wsum_scr, tri_scr, r1_scr, r2_scr, e1_scr, e2_scr):
    t = pl.program_id(0)

    @pl.when(t == 0)
    def _init():
        run_scr[...] = jnp.zeros_like(run_scr)
        wsum_scr[...] = jnp.zeros_like(wsum_scr)
        r = lax.broadcasted_iota(jnp.int32, (RBLK, RBLK), 0)
        c = lax.broadcasted_iota(jnp.int32, (RBLK, RBLK), 1)
        tri_scr[...] = (c < r).astype(jnp.bfloat16)

    x_cat = jnp.concatenate(
        [x_ref[...].astype(jnp.bfloat16),
         jnp.broadcast_to(pt_ref[0], (RBLK, N_C))], axis=1)
    logits = lax.dot_general(
        x_cat, w_g_ref[...], ((((1,), (0,))), ((), ())),
        preferred_element_type=jnp.float32)
    col = lax.broadcasted_iota(jnp.int32, logits.shape, 1)
    i1 = jnp.argmax(logits, axis=1, keepdims=True)
    v1 = jnp.max(logits, axis=1, keepdims=True)
    masked = jnp.where(col == i1, -jnp.inf, logits)
    i2 = jnp.argmax(masked, axis=1, keepdims=True)
    v2 = jnp.max(masked, axis=1, keepdims=True)
    e2v = jnp.exp(v2 - v1)
    g1 = 1.0 / (1.0 + e2v)
    g2 = e2v / (1.0 + e2v)

    oh1 = (col == i1).astype(jnp.float32)
    oh2 = (col == i2).astype(jnp.float32)
    ohs = oh1 + oh2
    w_blk = oh1 * g1 + oh2 * g2

    prior = run_scr[...]
    s_blk = lax.dot_general(
        tri_scr[...], ohs.astype(jnp.bfloat16), (((1,), (0,)), ((), ())),
        preferred_element_type=jnp.float32)
    base = prior + s_blk
    r1_scr[t] = jnp.sum(oh1 * base, axis=1, keepdims=True)
    r2_scr[t] = jnp.sum(oh2 * base, axis=1, keepdims=True)
    e1_scr[t] = i1
    e2_scr[t] = i2
    g1_ref[0] = g1
    g2_ref[0] = g2

    run_scr[...] += jnp.sum(ohs, axis=0, keepdims=True)
    wsum_scr[...] += jnp.sum(w_blk, axis=0, keepdims=True)

    @pl.when(t == N_RBLK - 1)
    def _fin():
        cnt = run_scr[...]
        padded = jnp.floor((cnt + (MBLK - 1)) * (1.0 / MBLK)) * MBLK
        ei = lax.broadcasted_iota(jnp.int32, (N_E, N_E), 0)
        ej = lax.broadcasted_iota(jnp.int32, (N_E, N_E), 1)
        tri8 = (ei < ej).astype(jnp.bfloat16)
        pad8 = jnp.broadcast_to(padded, (N_E, N_E)).astype(jnp.bfloat16)
        seg8 = lax.dot_general(
            pad8, tri8, (((1,), (0,)), ((), ())),
            preferred_element_type=jnp.float32)
        seg_row = seg8[0:1, :]

        total_used = jnp.sum(padded)
        brow = lax.broadcasted_iota(jnp.int32, (64, 1), 0).astype(jnp.float32)
        segb = jnp.broadcast_to(seg_row, (64, N_E))
        emap = (jnp.sum(
            (brow * MBLK >= segb).astype(jnp.float32), axis=1, keepdims=True)
            - 1.0).astype(jnp.int32)
        eob_ref[...] = jnp.where(brow * MBLK < total_used, emap, -1)

        colt = lax.broadcasted_iota(jnp.int32, (T_TOK, N_E), 1)
        segt = jnp.broadcast_to(seg_row, (T_TOK, N_E))
        e1a = e1_scr[...].reshape(T_TOK, 1)
        e2a = e2_scr[...].reshape(T_TOK, 1)
        s1 = jnp.sum(jnp.where(colt == e1a, segt, 0.0), axis=1, keepdims=True)
        s2 = jnp.sum(jnp.where(colt == e2a, segt, 0.0), axis=1, keepdims=True)
        p1 = (r1_scr[...].reshape(T_TOK, 1) + s1).astype(jnp.int32)
        p2 = (r2_scr[...].reshape(T_TOK, 1) + s2).astype(jnp.int32)
        p1_ref[...] = p1.reshape(N_RBLK, RBLK, 1)
        p2_ref[...] = p2.reshape(N_RBLK, RBLK, 1)

        def balance(v):
            m = jnp.mean(v)
            var = jnp.sum((v - m) ** 2) / (v.shape[-1] - 1)
            return var / (m * m + 1e-10)

        loss_ref[0, 0] = balance(wsum_scr[0, :]) + balance(cnt[0, :])


def _make_dispatch():
    mesh = plsc.VectorSubcoreMesh(core_axis_name="c", subcore_axis_name="s")

    @functools.partial(
        pl.kernel, mesh=mesh,
        out_type=jax.ShapeDtypeStruct((P_ROWS, N_C), jnp.float32),
        scratch_types=[
            pltpu.VMEM((HALF, N_C), jnp.float32),
            pltpu.VMEM((HALF, N_C), jnp.float32),
            pltpu.VMEM((HALF,), jnp.int32),
            pltpu.VMEM((HALF,), jnp.int32),
            pltpu.VMEM((HALF,), jnp.int32),
            pltpu.VMEM((HALF,), jnp.int32),
            pltpu.SemaphoreType.DMA,
            pltpu.SemaphoreType.DMA,
        ],
    )
    def dispatch(xt_hbm, p1_hbm, p2_hbm, xs_hbm,
                 rows_a, rows_b, p1a, p1b, p2a, p2b, sem1, sem2):
        wid = lax.axis_index("s") * 2 + lax.axis_index("c")
        base = wid * CHUNK
        pltpu.sync_copy(p1_hbm.at[wid, 0], p1a)
        pltpu.sync_copy(p1_hbm.at[wid, 1], p1b)
        pltpu.sync_copy(p2_hbm.at[wid, 0], p2a)
        pltpu.sync_copy(p2_hbm.at[wid, 1], p2b)
        pltpu.sync_copy(xt_hbm.at[pl.ds(base, HALF)], rows_a)
        c1 = pltpu.async_copy(rows_a, xs_hbm.at[p1a], sem1)
        c2 = pltpu.async_copy(rows_a, xs_hbm.at[p2a], sem1)
        pltpu.sync_copy(xt_hbm.at[pl.ds(base + HALF, HALF)], rows_b)
        c3 = pltpu.async_copy(rows_b, xs_hbm.at[p1b], sem2)
        c4 = pltpu.async_copy(rows_b, xs_hbm.at[p2b], sem2)
        c1.wait()
        c2.wait()
        c3.wait()
        c4.wait()

    return dispatch


def _mlp_kernel(eob_ref, xs_ref, fc1_w_ref, fc1_b_ref, fc2_w_ref, fc2_b_ref,
                o_ref):
    b = pl.program_id(0)
    e = eob_ref[b]

    @pl.when(e >= 0)
    def _body():
        x = xs_ref[...].astype(jnp.bfloat16)
        nch = 4
        hch = N_HID // nch
        acc = jnp.broadcast_to(fc2_b_ref[e], (MBLK, N_C))
        for k in range(nch):
            hk = lax.dot_general(
                x, fc1_w_ref[e, pl.ds(k * hch, hch)], (((1,), (1,)), ((), ())),
                preferred_element_type=jnp.float32) + fc1_b_ref[e, :, pl.ds(k * hch, hch)]
            gk = _gelu_exact(hk).astype(jnp.bfloat16)
            acc = acc + lax.dot_general(
                gk, fc2_w_ref[e, :, pl.ds(k * hch, hch)], (((1,), (1,)), ((), ())),
                preferred_element_type=jnp.float32)
        o_ref[...] = acc


def _run_mlp(eob, xs, fc1_w, fc1_b, fc2_w, fc2_b):
    resident = lambda b, s: (0, 0, 0)
    return pl.pallas_call(
        _mlp_kernel,
        grid_spec=pltpu.PrefetchScalarGridSpec(
            num_scalar_prefetch=1,
            grid=(N_MBLK,),
            in_specs=[
                pl.BlockSpec((MBLK, N_C), lambda b, s: (b, 0)),
                pl.BlockSpec((N_E, N_HID, N_C), resident),
                pl.BlockSpec((N_E, 1, N_HID), resident),
                pl.BlockSpec((N_E, N_C, N_HID), resident),
                pl.BlockSpec((N_E, 1, N_C), resident),
            ],
            out_specs=pl.BlockSpec((MBLK, N_C), lambda b, s: (b, 0)),
        ),
        out_shape=jax.ShapeDtypeStruct((P_ROWS, N_C), jnp.float32),
    )(eob, xs,
      fc1_w.astype(jnp.bfloat16), fc1_b.reshape(N_E, 1, N_HID),
      fc2_w.astype(jnp.bfloat16), fc2_b.reshape(N_E, 1, N_C))


def _make_gather():
    mesh = plsc.VectorSubcoreMesh(core_axis_name="c", subcore_axis_name="s")

    @functools.partial(
        pl.kernel, mesh=mesh,
        out_type=[
            jax.ShapeDtypeStruct((T_TOK, N_C), jnp.float32),
            jax.ShapeDtypeStruct((T_TOK, N_C), jnp.float32),
        ],
        scratch_types=[
            pltpu.VMEM((HALF, N_C), jnp.float32),
            pltpu.VMEM((HALF, N_C), jnp.float32),
            pltpu.VMEM((HALF, N_C), jnp.float32),
            pltpu.VMEM((HALF, N_C), jnp.float32),
            pltpu.VMEM((HALF,), jnp.int32),
            pltpu.VMEM((HALF,), jnp.int32),
            pltpu.VMEM((HALF,), jnp.int32),
            pltpu.VMEM((HALF,), jnp.int32),
            pltpu.SemaphoreType.DMA,
            pltpu.SemaphoreType.DMA,
            pltpu.SemaphoreType.DMA,
        ],
    )
    def gather(os_hbm, p1_hbm, p2_hbm, g1_hbm, g2_hbm,
               r1a, r2a, r1b, r2b, i1a, i1b, i2a, i2b, sem1, sem2, sem3):
        wid = lax.axis_index("s") * 2 + lax.axis_index("c")
        base = wid * CHUNK
        pltpu.sync_copy(p1_hbm.at[wid, 0], i1a)
        pltpu.sync_copy(p1_hbm.at[wid, 1], i1b)
        pltpu.sync_copy(p2_hbm.at[wid, 0], i2a)
        pltpu.sync_copy(p2_hbm.at[wid, 1], i2b)
        ga1 = pltpu.async_copy(os_hbm.at[i1a], r1a, sem1)
        ga2 = pltpu.async_copy(os_hbm.at[i2a], r2a, sem1)
        gb1 = pltpu.async_copy(os_hbm.at[i1b], r1b, sem2)
        gb2 = pltpu.async_copy(os_hbm.at[i2b], r2b, sem2)
        ga1.wait()
        ga2.wait()
        wa1 = pltpu.async_copy(r1a, g1_hbm.at[pl.ds(base, HALF)], sem3)
        wa2 = pltpu.async_copy(r2a, g2_hbm.at[pl.ds(base, HALF)], sem3)
        gb1.wait()
        gb2.wait()
        wb1 = pltpu.async_copy(r1b, g1_hbm.at[pl.ds(base + HALF, HALF)], sem3)
        wb2 = pltpu.async_copy(r2b, g2_hbm.at[pl.ds(base + HALF, HALF)], sem3)
        wa1.wait()
        wa2.wait()
        wb1.wait()
        wb2.wait()

    return gather


def _combine_kernel(o1_ref, o2_ref, g1_ref, g2_ref, y_ref):
    acc = jnp.exp(o1_ref[...]) * g1_ref[0] + jnp.exp(o2_ref[...]) * g2_ref[0]
    y_ref[...] = jnp.log(jnp.where(acc == 0.0, _EPS_FLOOR, acc))


def _run_combine(o1, o2, g1, g2):
    return pl.pallas_call(
        _combine_kernel,
        grid=(N_RBLK,),
        in_specs=[
            pl.BlockSpec((RBLK, N_C), lambda t: (t, 0)),
            pl.BlockSpec((RBLK, N_C), lambda t: (t, 0)),
            pl.BlockSpec((1, RBLK, 1), lambda t: (t, 0, 0)),
            pl.BlockSpec((1, RBLK, 1), lambda t: (t, 0, 0)),
        ],
        out_specs=pl.BlockSpec((RBLK, N_C), lambda t: (t, 0)),
        out_shape=jax.ShapeDtypeStruct((T_TOK, N_C), jnp.float32),
    )(o1, o2, g1, g2)


@jax.jit
def kernel(x, prompt, w_g, w_n, fc1_w, fc1_b, fc2_w, fc2_b):
    del w_n
    B, C, H, W = x.shape

    xt = jnp.transpose(x, (0, 2, 3, 1)).reshape(T_TOK, C)
    ptb = jnp.repeat(prompt.astype(jnp.bfloat16), N_RBLK // B, axis=0
                     ).reshape(N_RBLK, 1, C)

    p1, p2, g1, g2, eob, loss = _run_router_call(xt, ptb, w_g)

    w32 = lambda a: a.reshape(NW, 2, HALF)
    p1w, p2w = w32(p1), w32(p2)
    xs = _make_dispatch()(xt, p1w, p2w)

    o_s = _run_mlp(eob.reshape(64)[:N_MBLK], xs, fc1_w, fc1_b, fc2_w, fc2_b)

    o1, o2 = _make_gather()(o_s, p1w, p2w)

    y_flat = _run_combine(o1, o2, g1, g2)
    y = y_flat.reshape(B, H, W, C).transpose(0, 3, 1, 2)
    return y, loss


def _run_router_call(xt, ptb, w_g):
    whole = lambda t: (0, 0, 0)
    outs = pl.pallas_call(
        _router_kernel,
        grid=(N_RBLK,),
        in_specs=[
            pl.BlockSpec((RBLK, N_C), lambda t: (t, 0)),
            pl.BlockSpec((1, 1, N_C), lambda t: (t, 0, 0)),
            pl.BlockSpec((2 * N_C, N_E), lambda t: (0, 0)),
        ],
        out_specs=[
            pl.BlockSpec((N_RBLK, RBLK, 1), whole),
            pl.BlockSpec((N_RBLK, RBLK, 1), whole),
            pl.BlockSpec((1, RBLK, 1), lambda t: (t, 0, 0)),
            pl.BlockSpec((1, RBLK, 1), lambda t: (t, 0, 0)),
            pl.BlockSpec((64, 1), lambda t: (0, 0)),
            pl.BlockSpec(memory_space=pltpu.SMEM),
        ],
        out_shape=[
            jax.ShapeDtypeStruct((N_RBLK, RBLK, 1), jnp.int32),
            jax.ShapeDtypeStruct((N_RBLK, RBLK, 1), jnp.int32),
            jax.ShapeDtypeStruct((N_RBLK, RBLK, 1), jnp.float32),
            jax.ShapeDtypeStruct((N_RBLK, RBLK, 1), jnp.float32),
            jax.ShapeDtypeStruct((64, 1), jnp.int32),
            jax.ShapeDtypeStruct((1, 1), jnp.float32),
        ],
        scratch_shapes=[
            pltpu.VMEM((1, N_E), jnp.float32),
            pltpu.VMEM((1, N_E), jnp.float32),
            pltpu.VMEM((RBLK, RBLK), jnp.bfloat16),
            pltpu.VMEM((N_RBLK, RBLK, 1), jnp.float32),
            pltpu.VMEM((N_RBLK, RBLK, 1), jnp.float32),
            pltpu.VMEM((N_RBLK, RBLK, 1), jnp.int32),
            pltpu.VMEM((N_RBLK, RBLK, 1), jnp.int32),
        ],
    )(xt, ptb, w_g.astype(jnp.bfloat16))
    p1, p2, g1, g2, eob, loss = outs
    return p1, p2, g1, g2, eob, loss[0, 0]


# speedup vs baseline: 1.0724x; 1.0724x over previous
import functools

import jax
import jax.numpy as jnp
import numpy as np
from jax import lax
from jax.experimental import pallas as pl
from jax.experimental.pallas import tpu as pltpu
from jax.experimental.pallas import tpu_sc as plsc

_EPS_FLOOR = float(np.finfo(float).eps)

T_TOK = 4096
N_C = 384
N_E = 8
N_HID = 1536
RBLK = 512
N_RBLK = T_TOK // RBLK
MBLK = 512
N_MBLK = 2 * T_TOK // MBLK + N_E
P_ROWS = N_MBLK * MBLK
NW = 32
CHUNK = T_TOK // NW


def _gelu_exact(v):
    return v * 0.5 * (1.0 + lax.erf(v * np.float32(1.0 / np.sqrt(2.0))))


def _bf16_dot(a, b, dims):
    return lax.dot_general(
        a.astype(jnp.bfloat16), b.astype(jnp.bfloat16), (dims, ((), ())),
        preferred_element_type=jnp.float32)


def _router_kernel(x_ref, pt_ref, w_g_ref,
                   p1_ref, p2_ref, g1_ref, g2_ref, eob_ref, loss_ref,
                   run_scr, wsum_scr, tri_scr, r1_scr, r2_scr, e1_scr, e2_scr):
    t = pl.program_id(0)

    @pl.when(t == 0)
    def _init():
        run_scr[...] = jnp.zeros_like(run_scr)
        wsum_scr[...] = jnp.zeros_like(wsum_scr)
        r = lax.broadcasted_iota(jnp.int32, (RBLK, RBLK), 0)
        c = lax.broadcasted_iota(jnp.int32, (RBLK, RBLK), 1)
        tri_scr[...] = (c < r).astype(jnp.bfloat16)

    x_cat = jnp.concatenate(
        [x_ref[...].astype(jnp.bfloat16),
         jnp.broadcast_to(pt_ref[0], (RBLK, N_C))], axis=1)
    logits = lax.dot_general(
        x_cat, w_g_ref[...], ((((1,), (0,))), ((), ())),
        preferred_element_type=jnp.float32)
    col = lax.broadcasted_iota(jnp.int32, logits.shape, 1)
    i1 = jnp.argmax(logits, axis=1, keepdims=True)
    v1 = jnp.max(logits, axis=1, keepdims=True)
    masked = jnp.where(col == i1, -jnp.inf, logits)
    i2 = jnp.argmax(masked, axis=1, keepdims=True)
    v2 = jnp.max(masked, axis=1, keepdims=True)
    e2v = jnp.exp(v2 - v1)
    g1 = 1.0 / (1.0 + e2v)
    g2 = e2v / (1.0 + e2v)

    oh1 = (col == i1).astype(jnp.float32)
    oh2 = (col == i2).astype(jnp.float32)
    ohs = oh1 + oh2
    w_blk = oh1 * g1 + oh2 * g2

    prior = run_scr[...]
    s_blk = lax.dot_general(
        tri_scr[...], ohs.astype(jnp.bfloat16), (((1,), (0,)), ((), ())),
        preferred_element_type=jnp.float32)
    base = prior + s_blk
    r1_scr[t] = jnp.sum(oh1 * base, axis=1, keepdims=True)
    r2_scr[t] = jnp.sum(oh2 * base, axis=1, keepdims=True)
    e1_scr[t] = i1
    e2_scr[t] = i2
    g1_ref[0] = g1
    g2_ref[0] = g2

    run_scr[...] += jnp.sum(ohs, axis=0, keepdims=True)
    wsum_scr[...] += jnp.sum(w_blk, axis=0, keepdims=True)

    @pl.when(t == N_RBLK - 1)
    def _fin():
        cnt = run_scr[...]
        padded = jnp.floor((cnt + (MBLK - 1)) * (1.0 / MBLK)) * MBLK
        ei = lax.broadcasted_iota(jnp.int32, (N_E, N_E), 0)
        ej = lax.broadcasted_iota(jnp.int32, (N_E, N_E), 1)
        tri8 = (ei < ej).astype(jnp.bfloat16)
        pad8 = jnp.broadcast_to(padded, (N_E, N_E)).astype(jnp.bfloat16)
        seg8 = lax.dot_general(
            pad8, tri8, (((1,), (0,)), ((), ())),
            preferred_element_type=jnp.float32)
        seg_row = seg8[0:1, :]

        total_used = jnp.sum(padded)
        brow = lax.broadcasted_iota(jnp.int32, (64, 1), 0).astype(jnp.float32)
        segb = jnp.broadcast_to(seg_row, (64, N_E))
        emap = (jnp.sum(
            (brow * MBLK >= segb).astype(jnp.float32), axis=1, keepdims=True)
            - 1.0).astype(jnp.int32)
        eob_ref[...] = jnp.where(brow * MBLK < total_used, emap, -1)

        colt = lax.broadcasted_iota(jnp.int32, (T_TOK, N_E), 1)
        segt = jnp.broadcast_to(seg_row, (T_TOK, N_E))
        e1a = e1_scr[...].reshape(T_TOK, 1)
        e2a = e2_scr[...].reshape(T_TOK, 1)
        s1 = jnp.sum(jnp.where(colt == e1a, segt, 0.0), axis=1, keepdims=True)
        s2 = jnp.sum(jnp.where(colt == e2a, segt, 0.0), axis=1, keepdims=True)
        p1 = (r1_scr[...].reshape(T_TOK, 1) + s1).astype(jnp.int32)
        p2 = (r2_scr[...].reshape(T_TOK, 1) + s2).astype(jnp.int32)
        p1_ref[...] = p1.reshape(N_RBLK, RBLK, 1)
        p2_ref[...] = p2.reshape(N_RBLK, RBLK, 1)

        def balance(v):
            m = jnp.mean(v)
            var = jnp.sum((v - m) ** 2) / (v.shape[-1] - 1)
            return var / (m * m + 1e-10)

        loss_ref[0, 0] = balance(wsum_scr[0, :]) + balance(cnt[0, :])


def _make_dispatch():
    mesh = plsc.VectorSubcoreMesh(core_axis_name="c", subcore_axis_name="s")

    @functools.partial(
        pl.kernel, mesh=mesh,
        out_type=jax.ShapeDtypeStruct((P_ROWS, N_C), jnp.float32),
        scratch_types=[
            pltpu.VMEM((CHUNK, N_C), jnp.float32),
            pltpu.VMEM((CHUNK,), jnp.int32),
            pltpu.VMEM((CHUNK,), jnp.int32),
            pltpu.SemaphoreType.DMA,
            pltpu.SemaphoreType.DMA,
        ],
    )
    def dispatch(xt_hbm, p1_hbm, p2_hbm, xs_hbm,
                 rows_v, p1_v, p2_v, sem1, sem2):
        wid = lax.axis_index("s") * 2 + lax.axis_index("c")
        base = wid * CHUNK
        pltpu.sync_copy(xt_hbm.at[pl.ds(base, CHUNK)], rows_v)
        pltpu.sync_copy(p1_hbm.at[wid], p1_v)
        pltpu.sync_copy(p2_hbm.at[wid], p2_v)
        cp1 = pltpu.async_copy(rows_v, xs_hbm.at[p1_v], sem1)
        cp2 = pltpu.async_copy(rows_v, xs_hbm.at[p2_v], sem2)
        cp1.wait()
        cp2.wait()

    return dispatch


def _mlp_kernel(eob_ref, xs_ref, fc1_w_ref, fc1_b_ref, fc2_w_ref, fc2_b_ref,
                o_ref):
    b = pl.program_id(0)
    e = eob_ref[b]

    @pl.when(e >= 0)
    def _body():
        x = xs_ref[...].astype(jnp.bfloat16)
        h1 = lax.dot_general(
            x, fc1_w_ref[e], (((1,), (1,)), ((), ())),
            preferred_element_type=jnp.float32) + fc1_b_ref[e]
        h1 = _gelu_exact(h1).astype(jnp.bfloat16)
        o_ref[...] = lax.dot_general(
            h1, fc2_w_ref[e], (((1,), (1,)), ((), ())),
            preferred_element_type=jnp.float32) + fc2_b_ref[e]


def _run_mlp(eob, xs, fc1_w, fc1_b, fc2_w, fc2_b):
    resident = lambda b, s: (0, 0, 0)
    return pl.pallas_call(
        _mlp_kernel,
        grid_spec=pltpu.PrefetchScalarGridSpec(
            num_scalar_prefetch=1,
            grid=(N_MBLK,),
            in_specs=[
                pl.BlockSpec((MBLK, N_C), lambda b, s: (b, 0)),
                pl.BlockSpec((N_E, N_HID, N_C), resident),
                pl.BlockSpec((N_E, 1, N_HID), resident),
                pl.BlockSpec((N_E, N_C, N_HID), resident),
                pl.BlockSpec((N_E, 1, N_C), resident),
            ],
            out_specs=pl.BlockSpec((MBLK, N_C), lambda b, s: (b, 0)),
        ),
        out_shape=jax.ShapeDtypeStruct((P_ROWS, N_C), jnp.float32),
    )(eob, xs,
      fc1_w.astype(jnp.bfloat16), fc1_b.reshape(N_E, 1, N_HID),
      fc2_w.astype(jnp.bfloat16), fc2_b.reshape(N_E, 1, N_C))


def _make_gather():
    mesh = plsc.VectorSubcoreMesh(core_axis_name="c", subcore_axis_name="s")

    @functools.partial(
        pl.kernel, mesh=mesh,
        out_type=[
            jax.ShapeDtypeStruct((T_TOK, N_C), jnp.float32),
            jax.ShapeDtypeStruct((T_TOK, N_C), jnp.float32),
        ],
        scratch_types=[
            pltpu.VMEM((CHUNK, N_C), jnp.float32),
            pltpu.VMEM((CHUNK, N_C), jnp.float32),
            pltpu.VMEM((CHUNK,), jnp.int32),
            pltpu.VMEM((CHUNK,), jnp.int32),
            pltpu.SemaphoreType.DMA,
            pltpu.SemaphoreType.DMA,
        ],
    )
    def gather(os_hbm, p1_hbm, p2_hbm, g1_hbm, g2_hbm,
               rows1_v, rows2_v, i1_v, i2_v, sem1, sem2):
        wid = lax.axis_index("s") * 2 + lax.axis_index("c")
        base = wid * CHUNK
        pltpu.sync_copy(p1_hbm.at[wid], i1_v)
        pltpu.sync_copy(p2_hbm.at[wid], i2_v)
        cp1 = pltpu.async_copy(os_hbm.at[i1_v], rows1_v, sem1)
        cp2 = pltpu.async_copy(os_hbm.at[i2_v], rows2_v, sem2)
        cp1.wait()
        cp2.wait()
        pltpu.sync_copy(rows1_v, g1_hbm.at[pl.ds(base, CHUNK)])
        pltpu.sync_copy(rows2_v, g2_hbm.at[pl.ds(base, CHUNK)])

    return gather


def _combine_kernel(o1_ref, o2_ref, g1_ref, g2_ref, y_ref):
    acc = jnp.exp(o1_ref[...]) * g1_ref[0] + jnp.exp(o2_ref[...]) * g2_ref[0]
    y_ref[...] = jnp.log(jnp.where(acc == 0.0, _EPS_FLOOR, acc))


def _run_combine(o1, o2, g1, g2):
    return pl.pallas_call(
        _combine_kernel,
        grid=(N_RBLK,),
        in_specs=[
            pl.BlockSpec((RBLK, N_C), lambda t: (t, 0)),
            pl.BlockSpec((RBLK, N_C), lambda t: (t, 0)),
            pl.BlockSpec((1, RBLK, 1), lambda t: (t, 0, 0)),
            pl.BlockSpec((1, RBLK, 1), lambda t: (t, 0, 0)),
        ],
        out_specs=pl.BlockSpec((RBLK, N_C), lambda t: (t, 0)),
        out_shape=jax.ShapeDtypeStruct((T_TOK, N_C), jnp.float32),
    )(o1, o2, g1, g2)


@jax.jit
def kernel(x, prompt, w_g, w_n, fc1_w, fc1_b, fc2_w, fc2_b):
    del w_n
    B, C, H, W = x.shape

    xt = jnp.transpose(x, (0, 2, 3, 1)).reshape(T_TOK, C)
    ptb = jnp.repeat(prompt.astype(jnp.bfloat16), N_RBLK // B, axis=0
                     ).reshape(N_RBLK, 1, C)

    p1, p2, g1, g2, eob, loss = _run_router_call(xt, ptb, w_g)

    w32 = lambda a: a.reshape(NW, CHUNK)
    p1w, p2w = w32(p1), w32(p2)
    xs = _make_dispatch()(xt, p1w, p2w)

    o_s = _run_mlp(eob.reshape(64)[:N_MBLK], xs, fc1_w, fc1_b, fc2_w, fc2_b)

    o1, o2 = _make_gather()(o_s, p1w, p2w)

    y_flat = _run_combine(o1, o2, g1, g2)
    y = y_flat.reshape(B, H, W, C).transpose(0, 3, 1, 2)
    return y, loss


def _run_router_call(xt, ptb, w_g):
    whole = lambda t: (0, 0, 0)
    outs = pl.pallas_call(
        _router_kernel,
        grid=(N_RBLK,),
        in_specs=[
            pl.BlockSpec((RBLK, N_C), lambda t: (t, 0)),
            pl.BlockSpec((1, 1, N_C), lambda t: (t, 0, 0)),
            pl.BlockSpec((2 * N_C, N_E), lambda t: (0, 0)),
        ],
        out_specs=[
            pl.BlockSpec((N_RBLK, RBLK, 1), whole),
            pl.BlockSpec((N_RBLK, RBLK, 1), whole),
            pl.BlockSpec((1, RBLK, 1), lambda t: (t, 0, 0)),
            pl.BlockSpec((1, RBLK, 1), lambda t: (t, 0, 0)),
            pl.BlockSpec((64, 1), lambda t: (0, 0)),
            pl.BlockSpec(memory_space=pltpu.SMEM),
        ],
        out_shape=[
            jax.ShapeDtypeStruct((N_RBLK, RBLK, 1), jnp.int32),
            jax.ShapeDtypeStruct((N_RBLK, RBLK, 1), jnp.int32),
            jax.ShapeDtypeStruct((N_RBLK, RBLK, 1), jnp.float32),
            jax.ShapeDtypeStruct((N_RBLK, RBLK, 1), jnp.float32),
            jax.ShapeDtypeStruct((64, 1), jnp.int32),
            jax.ShapeDtypeStruct((1, 1), jnp.float32),
        ],
        scratch_shapes=[
            pltpu.VMEM((1, N_E), jnp.float32),
            pltpu.VMEM((1, N_E), jnp.float32),
            pltpu.VMEM((RBLK, RBLK), jnp.bfloat16),
            pltpu.VMEM((N_RBLK, RBLK, 1), jnp.float32),
            pltpu.VMEM((N_RBLK, RBLK, 1), jnp.float32),
            pltpu.VMEM((N_RBLK, RBLK, 1), jnp.int32),
            pltpu.VMEM((N_RBLK, RBLK, 1), jnp.int32),
        ],
    )(xt, ptb, w_g.astype(jnp.bfloat16))
    p1, p2, g1, g2, eob, loss = outs
    return p1, p2, g1, g2, eob, loss[0, 0]


# speedup vs baseline: 1.0996x; 1.0253x over previous
import functools

import jax
import jax.numpy as jnp
import numpy as np
from jax import lax
from jax.experimental import pallas as pl
from jax.experimental.pallas import tpu as pltpu
from jax.experimental.pallas import tpu_sc as plsc

_EPS_FLOOR = float(np.finfo(float).eps)

T_TOK = 4096
N_C = 384
N_E = 8
N_HID = 1536
RBLK = 1024
N_RBLK = T_TOK // RBLK
MBLK = 512
N_MBLK = 2 * T_TOK // MBLK + N_E
P_ROWS = N_MBLK * MBLK
NW = 32
CHUNK = T_TOK // NW


def _gelu_exact(v):
    return v * 0.5 * (1.0 + lax.erf(v * np.float32(1.0 / np.sqrt(2.0))))


def _bf16_dot(a, b, dims):
    return lax.dot_general(
        a.astype(jnp.bfloat16), b.astype(jnp.bfloat16), (dims, ((), ())),
        preferred_element_type=jnp.float32)


def _router_kernel(x_ref, pt_ref, w_g_ref,
                   p1_ref, p2_ref, g1_ref, g2_ref, eob_ref, loss_ref,
                   run_scr, wsum_scr, tri_scr, r1_scr, r2_scr, e1_scr, e2_scr):
    t = pl.program_id(0)

    @pl.when(t == 0)
    def _init():
        run_scr[...] = jnp.zeros_like(run_scr)
        wsum_scr[...] = jnp.zeros_like(wsum_scr)
        r = lax.broadcasted_iota(jnp.int32, (RBLK, RBLK), 0)
        c = lax.broadcasted_iota(jnp.int32, (RBLK, RBLK), 1)
        tri_scr[...] = (c < r).astype(jnp.bfloat16)

    x_cat = jnp.concatenate(
        [x_ref[...].astype(jnp.bfloat16),
         jnp.broadcast_to(pt_ref[0], (RBLK, N_C))], axis=1)
    logits = lax.dot_general(
        x_cat, w_g_ref[...], ((((1,), (0,))), ((), ())),
        preferred_element_type=jnp.float32)
    col = lax.broadcasted_iota(jnp.int32, logits.shape, 1)
    i1 = jnp.argmax(logits, axis=1, keepdims=True)
    v1 = jnp.max(logits, axis=1, keepdims=True)
    masked = jnp.where(col == i1, -jnp.inf, logits)
    i2 = jnp.argmax(masked, axis=1, keepdims=True)
    v2 = jnp.max(masked, axis=1, keepdims=True)
    e2v = jnp.exp(v2 - v1)
    g1 = 1.0 / (1.0 + e2v)
    g2 = e2v / (1.0 + e2v)

    oh1 = (col == i1).astype(jnp.float32)
    oh2 = (col == i2).astype(jnp.float32)
    ohs = oh1 + oh2
    w_blk = oh1 * g1 + oh2 * g2

    prior = run_scr[...]
    s_blk = lax.dot_general(
        tri_scr[...], ohs.astype(jnp.bfloat16), (((1,), (0,)), ((), ())),
        preferred_element_type=jnp.float32)
    base = prior + s_blk
    r1_scr[t] = jnp.sum(oh1 * base, axis=1, keepdims=True)
    r2_scr[t] = jnp.sum(oh2 * base, axis=1, keepdims=True)
    e1_scr[t] = i1
    e2_scr[t] = i2
    g1_ref[0] = g1
    g2_ref[0] = g2

    run_scr[...] += jnp.sum(ohs, axis=0, keepdims=True)
    wsum_scr[...] += jnp.sum(w_blk, axis=0, keepdims=True)

    @pl.when(t == N_RBLK - 1)
    def _fin():
        cnt = run_scr[...]
        padded = jnp.floor((cnt + (MBLK - 1)) * (1.0 / MBLK)) * MBLK
        ei = lax.broadcasted_iota(jnp.int32, (N_E, N_E), 0)
        ej = lax.broadcasted_iota(jnp.int32, (N_E, N_E), 1)
        tri8 = (ei < ej).astype(jnp.bfloat16)
        pad8 = jnp.broadcast_to(padded, (N_E, N_E)).astype(jnp.bfloat16)
        seg8 = lax.dot_general(
            pad8, tri8, (((1,), (0,)), ((), ())),
            preferred_element_type=jnp.float32)
        seg_row = seg8[0:1, :]

        total_used = jnp.sum(padded)
        brow = lax.broadcasted_iota(jnp.int32, (64, 1), 0).astype(jnp.float32)
        segb = jnp.broadcast_to(seg_row, (64, N_E))
        emap = (jnp.sum(
            (brow * MBLK >= segb).astype(jnp.float32), axis=1, keepdims=True)
            - 1.0).astype(jnp.int32)
        eob_ref[...] = jnp.where(brow * MBLK < total_used, emap, -1)

        colt = lax.broadcasted_iota(jnp.int32, (T_TOK, N_E), 1)
        segt = jnp.broadcast_to(seg_row, (T_TOK, N_E))
        e1a = e1_scr[...].reshape(T_TOK, 1)
        e2a = e2_scr[...].reshape(T_TOK, 1)
        s1 = jnp.sum(jnp.where(colt == e1a, segt, 0.0), axis=1, keepdims=True)
        s2 = jnp.sum(jnp.where(colt == e2a, segt, 0.0), axis=1, keepdims=True)
        p1 = (r1_scr[...].reshape(T_TOK, 1) + s1).astype(jnp.int32)
        p2 = (r2_scr[...].reshape(T_TOK, 1) + s2).astype(jnp.int32)
        p1_ref[...] = p1.reshape(N_RBLK, RBLK, 1)
        p2_ref[...] = p2.reshape(N_RBLK, RBLK, 1)

        def balance(v):
            m = jnp.mean(v)
            var = jnp.sum((v - m) ** 2) / (v.shape[-1] - 1)
            return var / (m * m + 1e-10)

        loss_ref[0, 0] = balance(wsum_scr[0, :]) + balance(cnt[0, :])


def _make_dispatch():
    mesh = plsc.VectorSubcoreMesh(core_axis_name="c", subcore_axis_name="s")

    @functools.partial(
        pl.kernel, mesh=mesh,
        out_type=jax.ShapeDtypeStruct((P_ROWS, N_C), jnp.float32),
        scratch_types=[
            pltpu.VMEM((CHUNK, N_C), jnp.float32),
            pltpu.VMEM((CHUNK,), jnp.int32),
            pltpu.VMEM((CHUNK,), jnp.int32),
            pltpu.SemaphoreType.DMA,
            pltpu.SemaphoreType.DMA,
        ],
    )
    def dispatch(xt_hbm, p1_hbm, p2_hbm, xs_hbm,
                 rows_v, p1_v, p2_v, sem1, sem2):
        wid = lax.axis_index("s") * 2 + lax.axis_index("c")
        base = wid * CHUNK
        pltpu.sync_copy(xt_hbm.at[pl.ds(base, CHUNK)], rows_v)
        pltpu.sync_copy(p1_hbm.at[wid], p1_v)
        pltpu.sync_copy(p2_hbm.at[wid], p2_v)
        cp1 = pltpu.async_copy(rows_v, xs_hbm.at[p1_v], sem1)
        cp2 = pltpu.async_copy(rows_v, xs_hbm.at[p2_v], sem2)
        cp1.wait()
        cp2.wait()

    return dispatch


def _mlp_kernel(eob_ref, xs_ref, fc1_w_ref, fc1_b_ref, fc2_w_ref, fc2_b_ref,
                o_ref):
    b = pl.program_id(0)
    e = eob_ref[b]

    @pl.when(e >= 0)
    def _body():
        x = xs_ref[...].astype(jnp.bfloat16)
        h1 = lax.dot_general(
            x, fc1_w_ref[e], (((1,), (1,)), ((), ())),
            preferred_element_type=jnp.float32) + fc1_b_ref[e]
        h1 = _gelu_exact(h1).astype(jnp.bfloat16)
        o_ref[...] = lax.dot_general(
            h1, fc2_w_ref[e], (((1,), (1,)), ((), ())),
            preferred_element_type=jnp.float32) + fc2_b_ref[e]


def _run_mlp(eob, xs, fc1_w, fc1_b, fc2_w, fc2_b):
    resident = lambda b, s: (0, 0, 0)
    return pl.pallas_call(
        _mlp_kernel,
        grid_spec=pltpu.PrefetchScalarGridSpec(
            num_scalar_prefetch=1,
            grid=(N_MBLK,),
            in_specs=[
                pl.BlockSpec((MBLK, N_C), lambda b, s: (b, 0)),
                pl.BlockSpec((N_E, N_HID, N_C), resident),
                pl.BlockSpec((N_E, 1, N_HID), resident),
                pl.BlockSpec((N_E, N_C, N_HID), resident),
                pl.BlockSpec((N_E, 1, N_C), resident),
            ],
            out_specs=pl.BlockSpec((MBLK, N_C), lambda b, s: (b, 0)),
        ),
        out_shape=jax.ShapeDtypeStruct((P_ROWS, N_C), jnp.float32),
    )(eob, xs,
      fc1_w.astype(jnp.bfloat16), fc1_b.reshape(N_E, 1, N_HID),
      fc2_w.astype(jnp.bfloat16), fc2_b.reshape(N_E, 1, N_C))


def _make_gather():
    mesh = plsc.VectorSubcoreMesh(core_axis_name="c", subcore_axis_name="s")

    @functools.partial(
        pl.kernel, mesh=mesh,
        out_type=[
            jax.ShapeDtypeStruct((T_TOK, N_C), jnp.float32),
            jax.ShapeDtypeStruct((T_TOK, N_C), jnp.float32),
        ],
        scratch_types=[
            pltpu.VMEM((CHUNK, N_C), jnp.float32),
            pltpu.VMEM((CHUNK, N_C), jnp.float32),
            pltpu.VMEM((CHUNK,), jnp.int32),
            pltpu.VMEM((CHUNK,), jnp.int32),
            pltpu.SemaphoreType.DMA,
            pltpu.SemaphoreType.DMA,
        ],
    )
    def gather(os_hbm, p1_hbm, p2_hbm, g1_hbm, g2_hbm,
               rows1_v, rows2_v, i1_v, i2_v, sem1, sem2):
        wid = lax.axis_index("s") * 2 + lax.axis_index("c")
        base = wid * CHUNK
        pltpu.sync_copy(p1_hbm.at[wid], i1_v)
        pltpu.sync_copy(p2_hbm.at[wid], i2_v)
        cp1 = pltpu.async_copy(os_hbm.at[i1_v], rows1_v, sem1)
        cp2 = pltpu.async_copy(os_hbm.at[i2_v], rows2_v, sem2)
        cp1.wait()
        cp2.wait()
        pltpu.sync_copy(rows1_v, g1_hbm.at[pl.ds(base, CHUNK)])
        pltpu.sync_copy(rows2_v, g2_hbm.at[pl.ds(base, CHUNK)])

    return gather


def _combine_kernel(o1_ref, o2_ref, g1_ref, g2_ref, y_ref):
    acc = jnp.exp(o1_ref[...]) * g1_ref[0] + jnp.exp(o2_ref[...]) * g2_ref[0]
    y_ref[...] = jnp.log(jnp.where(acc == 0.0, _EPS_FLOOR, acc))


def _run_combine(o1, o2, g1, g2):
    return pl.pallas_call(
        _combine_kernel,
        grid=(N_RBLK,),
        in_specs=[
            pl.BlockSpec((RBLK, N_C), lambda t: (t, 0)),
            pl.BlockSpec((RBLK, N_C), lambda t: (t, 0)),
            pl.BlockSpec((1, RBLK, 1), lambda t: (t, 0, 0)),
            pl.BlockSpec((1, RBLK, 1), lambda t: (t, 0, 0)),
        ],
        out_specs=pl.BlockSpec((RBLK, N_C), lambda t: (t, 0)),
        out_shape=jax.ShapeDtypeStruct((T_TOK, N_C), jnp.float32),
    )(o1, o2, g1, g2)


@jax.jit
def kernel(x, prompt, w_g, w_n, fc1_w, fc1_b, fc2_w, fc2_b):
    del w_n
    B, C, H, W = x.shape

    xt = jnp.transpose(x, (0, 2, 3, 1)).reshape(T_TOK, C)
    ptb = jnp.repeat(prompt.astype(jnp.bfloat16), N_RBLK // B, axis=0
                     ).reshape(N_RBLK, 1, C)

    p1, p2, g1, g2, eob, loss = _run_router_call(xt, ptb, w_g)

    w32 = lambda a: a.reshape(NW, CHUNK)
    p1w, p2w = w32(p1), w32(p2)
    xs = _make_dispatch()(xt, p1w, p2w)

    o_s = _run_mlp(eob.reshape(64)[:N_MBLK], xs, fc1_w, fc1_b, fc2_w, fc2_b)

    o1, o2 = _make_gather()(o_s, p1w, p2w)

    y_flat = _run_combine(o1, o2, g1, g2)
    y = y_flat.reshape(B, H, W, C).transpose(0, 3, 1, 2)
    return y, loss


def _run_router_call(xt, ptb, w_g):
    whole = lambda t: (0, 0, 0)
    outs = pl.pallas_call(
        _router_kernel,
        grid=(N_RBLK,),
        in_specs=[
            pl.BlockSpec((RBLK, N_C), lambda t: (t, 0)),
            pl.BlockSpec((1, 1, N_C), lambda t: (t, 0, 0)),
            pl.BlockSpec((2 * N_C, N_E), lambda t: (0, 0)),
        ],
        out_specs=[
            pl.BlockSpec((N_RBLK, RBLK, 1), whole),
            pl.BlockSpec((N_RBLK, RBLK, 1), whole),
            pl.BlockSpec((1, RBLK, 1), lambda t: (t, 0, 0)),
            pl.BlockSpec((1, RBLK, 1), lambda t: (t, 0, 0)),
            pl.BlockSpec((64, 1), lambda t: (0, 0)),
            pl.BlockSpec(memory_space=pltpu.SMEM),
        ],
        out_shape=[
            jax.ShapeDtypeStruct((N_RBLK, RBLK, 1), jnp.int32),
            jax.ShapeDtypeStruct((N_RBLK, RBLK, 1), jnp.int32),
            jax.ShapeDtypeStruct((N_RBLK, RBLK, 1), jnp.float32),
            jax.ShapeDtypeStruct((N_RBLK, RBLK, 1), jnp.float32),
            jax.ShapeDtypeStruct((64, 1), jnp.int32),
            jax.ShapeDtypeStruct((1, 1), jnp.float32),
        ],
        scratch_shapes=[
            pltpu.VMEM((1, N_E), jnp.float32),
            pltpu.VMEM((1, N_E), jnp.float32),
            pltpu.VMEM((RBLK, RBLK), jnp.bfloat16),
            pltpu.VMEM((N_RBLK, RBLK, 1), jnp.float32),
            pltpu.VMEM((N_RBLK, RBLK, 1), jnp.float32),
            pltpu.VMEM((N_RBLK, RBLK, 1), jnp.int32),
            pltpu.VMEM((N_RBLK, RBLK, 1), jnp.int32),
        ],
    )(xt, ptb, w_g.astype(jnp.bfloat16))
    p1, p2, g1, g2, eob, loss = outs
    return p1, p2, g1, g2, eob, loss[0, 0]
